# Initial kernel scaffold; baseline (speedup 1.0000x reference)
import functools

import jax
import jax.numpy as jnp
from jax import lax
from jax.experimental import pallas as pl
from jax.experimental.pallas import tpu as pltpu
from jax.experimental.pallas import tpu_sc as plsc

NN = 10000
NPAD = 10240
EB = 128
NBATCH = 42
EPT = EB * NBATCH
NTILES = 32
ETP = EPT * NTILES
DUMMY = 10100
SLICE = NPAD // 16
EPS = 1e-5
NC = 2


def _mm_alpha_body(x_ref, w_ref, a_ref, h_ref, al_ref):
    h = jnp.dot(x_ref[...], w_ref[...], preferred_element_type=jnp.float32)
    h_ref[0] = h

    @pl.when(pl.program_id(1) == 0)
    def _():
        al_ref[...] = jnp.zeros_like(al_ref)

    al_ref[...] += jnp.dot(h, a_ref[0], preferred_element_type=jnp.float32)


def _mm_alpha(xp, W, aall, C):
    K = xp.shape[1]
    BM = 512
    return pl.pallas_call(
        _mm_alpha_body,
        grid=(NPAD // BM, C),
        in_specs=[
            pl.BlockSpec((BM, K), lambda m, c: (m, 0)),
            pl.BlockSpec((K, 128), lambda m, c: (0, c)),
            pl.BlockSpec((1, 128, 128), lambda m, c: (c, 0, 0)),
        ],
        out_specs=[
            pl.BlockSpec((1, BM, 128), lambda m, c: (c, m, 0)),
            pl.BlockSpec((BM, 128), lambda m, c: (m, 0)),
        ],
        out_shape=[
            jax.ShapeDtypeStruct((C, NPAD, 128), jnp.float32),
            jax.ShapeDtypeStruct((NPAD, 128), jnp.float32),
        ],
    )(xp, W, aall)


def _bd_body(a_ref, bd_ref):
    a = a_ref[...]
    ms = jnp.max(a)
    v = a + ms
    bd_ref[...] = jnp.maximum(v, 0.2 * v)


def _bd(al):
    return pl.pallas_call(
        _bd_body,
        out_shape=jax.ShapeDtypeStruct((NPAD, 128), jnp.float32),
    )(al)


def _combine_body(op_ref, b_ref, g_ref, be_ref, rm_ref, rv_ref, o_ref, *, act, bmc):
    o = op_ref[0, 0] + op_ref[1, 0]
    o = o + b_ref[0]
    if act == "bn_elu":
        o = (o - rm_ref[0]) / jnp.sqrt(rv_ref[0] + EPS) * g_ref[0] + be_ref[0]
        o = jnp.where(o > 0, o, jnp.expm1(o))
        rows = lax.broadcasted_iota(jnp.int32, o.shape, 0) + pl.program_id(1) * bmc
        o = jnp.where(rows < NN, o, 0.0)
    else:
        mx = jnp.max(o, axis=1, keepdims=True)
        s = o - mx
        o = s - jnp.log(jnp.sum(jnp.exp(s), axis=1, keepdims=True))
    o_ref[...] = o


def _combine(outp, bias, g, be, rm, rv, C, act):
    BMC = 512
    body = functools.partial(_combine_body, act=act, bmc=BMC)
    return pl.pallas_call(
        body,
        grid=(C, NPAD // BMC),
        in_specs=[
            pl.BlockSpec((2, 1, BMC, 128), lambda c, m: (0, c, m, 0)),
            pl.BlockSpec((1, 128), lambda c, m: (c, 0)),
            pl.BlockSpec((1, 128), lambda c, m: (c, 0)),
            pl.BlockSpec((1, 128), lambda c, m: (c, 0)),
            pl.BlockSpec((1, 128), lambda c, m: (c, 0)),
            pl.BlockSpec((1, 128), lambda c, m: (c, 0)),
        ],
        out_specs=pl.BlockSpec((BMC, 128), lambda c, m: (m, c)),
        out_shape=jax.ShapeDtypeStruct((NPAD, C * 128), jnp.float32),
    )(outp, bias, g, be, rm, rv)


def _make_sc_a(H):
    mesh = plsc.VectorSubcoreMesh(core_axis_name="c", subcore_axis_name="s")
    out_type = [jax.ShapeDtypeStruct((NTILES, NBATCH, EB), jnp.float32) for _ in range(H)]
    out_type.append(jax.ShapeDtypeStruct((NC, H, NPAD), jnp.float32))
    scratch = [
        pltpu.VMEM((NBATCH, EB), jnp.int32),
        pltpu.VMEM((NBATCH, EB), jnp.int32),
        pltpu.VMEM((EB,), jnp.float32),
        pltpu.VMEM((EB,), jnp.float32),
        pltpu.VMEM((EB,), jnp.float32),
        pltpu.VMEM((EB,), jnp.float32),
        pltpu.VMEM((SLICE,), jnp.float32),
    ] + [pltpu.VMEM_SHARED((NPAD,), jnp.float32) for _ in range(H)]

    def body(*refs):
        src_h, dst_h = refs[0], refs[1]
        as_t = refs[2:2 + H]
        ad_t = refs[2 + H:2 + 2 * H]
        bd_t = refs[2 + 2 * H:2 + 3 * H]
        t_out = refs[2 + 3 * H:2 + 4 * H]
        s_out = refs[2 + 4 * H]
        src_v, dst_v, bs, bdv, bb, tb, zb = refs[3 + 4 * H:10 + 4 * H]
        s_sh = refs[10 + 4 * H:10 + 5 * H]
        core = lax.axis_index("c")
        sid = lax.axis_index("s")
        wid = sid * NC + core
        pltpu.sync_copy(src_h.at[wid], src_v)
        pltpu.sync_copy(dst_h.at[wid], dst_v)
        for j in range(SLICE // 16):
            zb[pl.ds(j * 16, 16)] = jnp.zeros((16,), jnp.float32)
        for h in range(H):
            pltpu.sync_copy(zb, s_sh[h].at[pl.ds(sid * SLICE, SLICE)])
        plsc.subcore_barrier()

        def bstep(b, carry):
            for h in range(H):
                pltpu.sync_copy(as_t[h].at[src_v.at[b]], bs)
                pltpu.sync_copy(ad_t[h].at[dst_v.at[b]], bdv)
                pltpu.sync_copy(bd_t[h].at[dst_v.at[b]], bb)
                for k in range(EB // 16):
                    sl = pl.ds(k * 16, 16)
                    e = bs[sl] + bdv[sl]
                    e = jnp.maximum(e, 0.2 * e)
                    tb[sl] = jnp.exp(e - bb[sl])
                pltpu.sync_copy(tb, t_out[h].at[wid, b])
                pltpu.sync_copy(tb, s_sh[h].at[dst_v.at[b]], add=True)
            return carry

        lax.fori_loop(0, NBATCH, bstep, 0)
        plsc.subcore_barrier()
        for h in range(H):
            pltpu.sync_copy(
                s_sh[h].at[pl.ds(sid * SLICE, SLICE)],
                s_out.at[core, h, pl.ds(sid * SLICE, SLICE)],
            )

    return pl.kernel(body, out_type=out_type, mesh=mesh, scratch_types=scratch)


def _make_sc_b(H, C):
    mesh = plsc.VectorSubcoreMesh(core_axis_name="c", subcore_axis_name="s")
    out_type = jax.ShapeDtypeStruct((NC, C, NPAD, 128), jnp.float32)
    scratch = [
        pltpu.VMEM((NBATCH, EB), jnp.int32),
        pltpu.VMEM((NBATCH, EB), jnp.int32),
        pltpu.VMEM((H, NBATCH, EB), jnp.float32),
        pltpu.VMEM((EB,), jnp.float32),
        pltpu.VMEM((EB,), jnp.float32),
        pltpu.VMEM((EB,), jnp.float32),
        pltpu.VMEM((EB, 128), jnp.float32),
        pltpu.VMEM((128, 128), jnp.float32),
        pltpu.VMEM_SHARED((NPAD, 128), jnp.float32),
    ]

    def body(*refs):
        src_h, dst_h = refs[0], refs[1]
        t_in = refs[2:2 + H]
        s0 = refs[2 + H:2 + 2 * H]
        s1 = refs[2 + 2 * H:2 + 3 * H]
        hcs = refs[2 + 3 * H:2 + 3 * H + C]
        outp = refs[2 + 3 * H + C]
        src_v, dst_v, rv, tb, s0b, s1b, gbuf, zb, acc = refs[3 + 3 * H + C:]
        core = lax.axis_index("c")
        sid = lax.axis_index("s")
        wid = sid * NC + core
        pltpu.sync_copy(src_h.at[wid], src_v)
        pltpu.sync_copy(dst_h.at[wid], dst_v)

        def rstep(b, carry):
            for h in range(H):
                pltpu.sync_copy(t_in[h].at[wid, b], tb)
                pltpu.sync_copy(s0[h].at[dst_v.at[b]], s0b)
                pltpu.sync_copy(s1[h].at[dst_v.at[b]], s1b)
                for k in range(EB // 16):
                    sl = pl.ds(k * 16, 16)
                    rv[h, b, sl] = tb[sl] / (s0b[sl] + s1b[sl] + 1e-16)
            return carry

        lax.fori_loop(0, NBATCH, rstep, 0)

        def zstep(j, carry):
            for k in range(8):
                zb[j, pl.ds(k * 16, 16)] = jnp.zeros((16,), jnp.float32)
            return carry

        lax.fori_loop(0, 128, zstep, 0)

        for c in range(C):
            hd = c // (C // H)
            for j in range(SLICE // 128):
                pltpu.sync_copy(zb, acc.at[pl.ds(sid * SLICE + j * 128, 128)])
            plsc.subcore_barrier()

            def estep(b, carry, c=c, hd=hd):
                pltpu.sync_copy(hcs[c].at[src_v.at[b]], gbuf)

                def scale(i, c3):
                    s = rv[hd, b, i]
                    for k in range(8):
                        sl = pl.ds(k * 16, 16)
                        gbuf[i, sl] = gbuf[i, sl] * s
                    return c3

                lax.fori_loop(0, EB, scale, 0)
                pltpu.sync_copy(gbuf, acc.at[dst_v.at[b]], add=True)
                return carry

            lax.fori_loop(0, NBATCH, estep, 0)
            plsc.subcore_barrier()
            pltpu.sync_copy(
                acc.at[pl.ds(sid * SLICE, SLICE)],
                outp.at[core, c, pl.ds(sid * SLICE, SLICE)],
            )

    return pl.kernel(body, out_type=out_type, mesh=mesh, scratch_types=scratch)


def _build_aall(a_s, a_d, C, H):
    per = C // H
    cols = []
    for c in range(C):
        hd = c // per
        o = c % per
        blk = jnp.zeros((128, 128), jnp.float32)
        blk = blk.at[:, hd].set(a_s[hd, o * 128:(o + 1) * 128])
        blk = blk.at[:, H + hd].set(a_d[hd, o * 128:(o + 1) * 128])
        cols.append(blk)
    return jnp.stack(cols)


def _gat_layer(xin, src, dst, W, a_s, a_d, H, C, sc_a, sc_b):
    aall = _build_aall(a_s, a_d, C, H)
    hc, al = _mm_alpha(xin, W, aall, C)
    bdw = _bd(al)
    as_t = [al[:, h] for h in range(H)]
    ad_t = [al[:, H + h] for h in range(H)]
    bd_t = [bdw[:, H + h] for h in range(H)]
    outs = sc_a(src, dst, *as_t, *ad_t, *bd_t)
    t_list = list(outs[:H])
    s_p = outs[H]
    s0 = [s_p[0, h] for h in range(H)]
    s1 = [s_p[1, h] for h in range(H)]
    hcs = [hc[c] for c in range(C)]
    outp = sc_b(src, dst, *t_list, *s0, *s1, *hcs)
    return outp


def kernel(x, edge_index, W1, a_src1, a_dst1, b1, g1, be1, rm1, rv1,
           W2, a_src2, a_dst2, b2, g2, be2, rm2, rv2,
           W3, a_src3, a_dst3, b3):
    loop = jnp.arange(NN, dtype=jnp.int32)
    src = jnp.concatenate([edge_index[0].astype(jnp.int32), loop])
    dst = jnp.concatenate([edge_index[1].astype(jnp.int32), loop])
    npad_e = ETP - src.shape[0]
    src = jnp.concatenate([src, jnp.zeros((npad_e,), jnp.int32)])
    dst = jnp.concatenate([dst, jnp.full((npad_e,), DUMMY, jnp.int32)])
    src = src.reshape(NTILES, NBATCH, EB)
    dst = dst.reshape(NTILES, NBATCH, EB)
    xp = jnp.pad(x, ((0, NPAD - NN), (0, 0)))

    sc_a4 = _make_sc_a(4)
    sc_b4 = _make_sc_b(4, 16)
    sc_a1 = _make_sc_a(1)
    sc_b1_4 = _make_sc_b(1, 4)
    sc_b1_1 = _make_sc_b(1, 1)

    zero = jnp.zeros((128,), jnp.float32)
    one = jnp.ones((128,), jnp.float32)

    outp1 = _gat_layer(xp, src, dst, W1, a_src1, a_dst1, 4, 16, sc_a4, sc_b4)
    y1 = _combine(outp1, b1.reshape(16, 128), g1.reshape(16, 128),
                  be1.reshape(16, 128), rm1.reshape(16, 128),
                  rv1.reshape(16, 128), 16, "bn_elu")

    outp2 = _gat_layer(y1, src, dst, W2, a_src2, a_dst2, 1, 4, sc_a1, sc_b1_4)
    y2 = _combine(outp2, b2.reshape(4, 128), g2.reshape(4, 128),
                  be2.reshape(4, 128), rm2.reshape(4, 128),
                  rv2.reshape(4, 128), 4, "bn_elu")

    outp3 = _gat_layer(y2, src, dst, W3, a_src3, a_dst3, 1, 1, sc_a1, sc_b1_1)
    y3 = _combine(outp3, b3.reshape(1, 128), one.reshape(1, 128),
                  zero.reshape(1, 128), zero.reshape(1, 128),
                  one.reshape(1, 128), 1, "logsoftmax")
    return y3[:NN]


# speedup vs baseline: 5.7310x; 5.7310x over previous
import functools

import jax
import jax.numpy as jnp
from jax import lax
from jax.experimental import pallas as pl
from jax.experimental.pallas import tpu as pltpu
from jax.experimental.pallas import tpu_sc as plsc

NN = 10000
NPAD = 10240
EB = 128
NBATCH = 42
EPT = EB * NBATCH
NTILES = 32
ETP = EPT * NTILES
DUMMY = 10100
SLICE = NPAD // 16
EPS = 1e-5
NC = 2


def _mm_alpha_body(x_ref, w_ref, a_ref, h_ref, al_ref):
    h = jnp.dot(x_ref[...], w_ref[...], preferred_element_type=jnp.float32)
    h_ref[0] = h

    @pl.when(pl.program_id(1) == 0)
    def _():
        al_ref[...] = jnp.zeros_like(al_ref)

    al_ref[...] += jnp.dot(h, a_ref[0], preferred_element_type=jnp.float32)


def _mm_alpha(xp, W, aall, C):
    K = xp.shape[1]
    BM = 512
    return pl.pallas_call(
        _mm_alpha_body,
        grid=(NPAD // BM, C),
        in_specs=[
            pl.BlockSpec((BM, K), lambda m, c: (m, 0)),
            pl.BlockSpec((K, 128), lambda m, c: (0, c)),
            pl.BlockSpec((1, 128, 128), lambda m, c: (c, 0, 0)),
        ],
        out_specs=[
            pl.BlockSpec((1, BM, 128), lambda m, c: (c, m, 0)),
            pl.BlockSpec((BM, 128), lambda m, c: (m, 0)),
        ],
        out_shape=[
            jax.ShapeDtypeStruct((C, NPAD, 128), jnp.float32),
            jax.ShapeDtypeStruct((NPAD, 128), jnp.float32),
        ],
    )(xp, W, aall)


def _bd_body(a_ref, bd_ref):
    a = a_ref[...]
    ms = jnp.max(a)
    v = a + ms
    bd_ref[...] = jnp.maximum(v, 0.2 * v)


def _bd(al):
    return pl.pallas_call(
        _bd_body,
        out_shape=jax.ShapeDtypeStruct((NPAD, 128), jnp.float32),
    )(al)


def _combine_body(op_ref, b_ref, g_ref, be_ref, rm_ref, rv_ref, o_ref, *, act, bmc):
    o = op_ref[0, 0] + op_ref[1, 0]
    o = o + b_ref[0, 0]
    if act == "bn_elu":
        o = (o - rm_ref[0, 0]) / jnp.sqrt(rv_ref[0, 0] + EPS) * g_ref[0, 0] + be_ref[0, 0]
        o = jnp.where(o > 0, o, jnp.exp(o) - 1.0)
        rows = lax.broadcasted_iota(jnp.int32, o.shape, 0) + pl.program_id(1) * bmc
        o = jnp.where(rows < NN, o, 0.0)
    else:
        mx = jnp.max(o, axis=1, keepdims=True)
        s = o - mx
        o = s - jnp.log(jnp.sum(jnp.exp(s), axis=1, keepdims=True))
    o_ref[...] = o


def _combine(outp, bias, g, be, rm, rv, C, act):
    BMC = 512
    body = functools.partial(_combine_body, act=act, bmc=BMC)
    return pl.pallas_call(
        body,
        grid=(C, NPAD // BMC),
        in_specs=[
            pl.BlockSpec((2, 1, BMC, 128), lambda c, m: (0, c, m, 0)),
            pl.BlockSpec((1, 1, 128), lambda c, m: (c, 0, 0)),
            pl.BlockSpec((1, 1, 128), lambda c, m: (c, 0, 0)),
            pl.BlockSpec((1, 1, 128), lambda c, m: (c, 0, 0)),
            pl.BlockSpec((1, 1, 128), lambda c, m: (c, 0, 0)),
            pl.BlockSpec((1, 1, 128), lambda c, m: (c, 0, 0)),
        ],
        out_specs=pl.BlockSpec((BMC, 128), lambda c, m: (m, c)),
        out_shape=jax.ShapeDtypeStruct((NPAD, C * 128), jnp.float32),
    )(outp.reshape(2, C, NPAD, 128), bias.reshape(C, 1, 128), g.reshape(C, 1, 128),
      be.reshape(C, 1, 128), rm.reshape(C, 1, 128), rv.reshape(C, 1, 128))


def _make_sc_a(H):
    mesh = plsc.VectorSubcoreMesh(core_axis_name="c", subcore_axis_name="s")
    out_type = [jax.ShapeDtypeStruct((NTILES, NBATCH, EB), jnp.float32) for _ in range(H)]
    out_type.append(jax.ShapeDtypeStruct((NC, H, NPAD), jnp.float32))
    scratch = [
        pltpu.VMEM((NBATCH, EB), jnp.int32),
        pltpu.VMEM((NBATCH, EB), jnp.int32),
        pltpu.VMEM((EB,), jnp.float32),
        pltpu.VMEM((EB,), jnp.float32),
        pltpu.VMEM((EB,), jnp.float32),
        pltpu.VMEM((EB,), jnp.float32),
        pltpu.VMEM((SLICE,), jnp.float32),
    ] + [pltpu.VMEM_SHARED((NPAD,), jnp.float32) for _ in range(H)]

    def body(*refs):
        src_h, dst_h = refs[0], refs[1]
        as_t = refs[2:2 + H]
        ad_t = refs[2 + H:2 + 2 * H]
        bd_t = refs[2 + 2 * H:2 + 3 * H]
        t_out = refs[2 + 3 * H:2 + 4 * H]
        s_out = refs[2 + 4 * H]
        src_v, dst_v, bs, bdv, bb, tb, zb = refs[3 + 4 * H:10 + 4 * H]
        s_sh = refs[10 + 4 * H:10 + 5 * H]
        core = lax.axis_index("c")
        sid = lax.axis_index("s")
        wid = sid * NC + core
        pltpu.sync_copy(src_h.at[wid], src_v)
        pltpu.sync_copy(dst_h.at[wid], dst_v)
        for j in range(SLICE // 16):
            zb[pl.ds(j * 16, 16)] = jnp.zeros((16,), jnp.float32)
        for h in range(H):
            pltpu.sync_copy(zb, s_sh[h].at[pl.ds(sid * SLICE, SLICE)])
        plsc.subcore_barrier()

        def bstep(b, carry):
            for h in range(H):
                pltpu.sync_copy(as_t[h].at[src_v.at[b]], bs)
                pltpu.sync_copy(ad_t[h].at[dst_v.at[b]], bdv)
                pltpu.sync_copy(bd_t[h].at[dst_v.at[b]], bb)
                for k in range(EB // 16):
                    sl = pl.ds(k * 16, 16)
                    e = bs[sl] + bdv[sl]
                    e = jnp.maximum(e, 0.2 * e)
                    tb[sl] = jnp.exp(e - bb[sl])
                pltpu.sync_copy(tb, t_out[h].at[wid, b])
                pltpu.sync_copy(tb, s_sh[h].at[dst_v.at[b]], add=True)
            return carry

        lax.fori_loop(0, NBATCH, bstep, 0)
        plsc.subcore_barrier()
        for h in range(H):
            pltpu.sync_copy(
                s_sh[h].at[pl.ds(sid * SLICE, SLICE)],
                s_out.at[core, h, pl.ds(sid * SLICE, SLICE)],
            )

    return pl.kernel(body, out_type=out_type, mesh=mesh, scratch_types=scratch)


def _make_sc_b(H, C):
    mesh = plsc.VectorSubcoreMesh(core_axis_name="c", subcore_axis_name="s")
    out_type = jax.ShapeDtypeStruct((NC, C, NPAD, 128), jnp.float32)
    scratch = [
        pltpu.VMEM((NBATCH, EB), jnp.int32),
        pltpu.VMEM((NBATCH, EB), jnp.int32),
        pltpu.VMEM((NBATCH * EB,), jnp.float32),
        pltpu.VMEM((EB,), jnp.float32),
        pltpu.VMEM((EB,), jnp.float32),
        pltpu.VMEM((EB,), jnp.float32),
        pltpu.VMEM((EB, 128), jnp.float32),
        pltpu.VMEM_SHARED((NPAD, 128), jnp.float32),
    ]

    def body(*refs):
        src_h, dst_h = refs[0], refs[1]
        t_in = refs[2:2 + H]
        s0 = refs[2 + H:2 + 2 * H]
        s1 = refs[2 + 2 * H:2 + 3 * H]
        hcs = refs[2 + 3 * H:2 + 3 * H + C]
        outp = refs[2 + 3 * H + C]
        src_v, dst_v, rv, tb, s0b, s1b, gbuf, acc = refs[3 + 3 * H + C:]
        core = lax.axis_index("c")
        sid = lax.axis_index("s")
        wid = sid * NC + core
        pltpu.sync_copy(src_h.at[wid], src_v)
        pltpu.sync_copy(dst_h.at[wid], dst_v)

        def zero_gbuf():
            def zstep(j, carry):
                for k in range(8):
                    gbuf[j, pl.ds(k * 16, 16)] = jnp.zeros((16,), jnp.float32)
                return carry

            lax.fori_loop(0, EB, zstep, 0)

        for c in range(C):
            hd = c // (C // H)
            if c % (C // H) == 0:
                def rstep(b, carry, hd=hd):
                    pltpu.sync_copy(t_in[hd].at[wid, b], tb)
                    pltpu.sync_copy(s0[hd].at[dst_v.at[b]], s0b)
                    pltpu.sync_copy(s1[hd].at[dst_v.at[b]], s1b)
                    for k in range(EB // 16):
                        sl = pl.ds(k * 16, 16)
                        rv[pl.ds(b * EB + k * 16, 16)] = (
                            tb[sl] / (s0b[sl] + s1b[sl] + 1e-16))
                    return carry

                lax.fori_loop(0, NBATCH, rstep, 0)
            zero_gbuf()
            for j in range(SLICE // 128):
                pltpu.sync_copy(gbuf, acc.at[pl.ds(sid * SLICE + j * 128, 128)])
            plsc.subcore_barrier()

            def estep(b, carry, c=c):
                pltpu.sync_copy(hcs[c].at[src_v.at[b]], gbuf)

                def scale(g, c3):
                    rvec = rv[pl.ds(b * EB + g * 16, 16)]
                    for j in range(16):
                        s = rvec[j]
                        for k in range(8):
                            sl = pl.ds(k * 16, 16)
                            gbuf[g * 16 + j, sl] = gbuf[g * 16 + j, sl] * s
                    return c3

                lax.fori_loop(0, EB // 16, scale, 0)
                pltpu.sync_copy(gbuf, acc.at[dst_v.at[b]], add=True)
                return carry

            lax.fori_loop(0, NBATCH, estep, 0)
            plsc.subcore_barrier()
            pltpu.sync_copy(
                acc.at[pl.ds(sid * SLICE, SLICE)],
                outp.at[core, c, pl.ds(sid * SLICE, SLICE)],
            )

    return pl.kernel(body, out_type=out_type, mesh=mesh, scratch_types=scratch)


def _build_aall(a_s, a_d, C, H):
    per = C // H
    cols = []
    for c in range(C):
        hd = c // per
        o = c % per
        blk = jnp.zeros((128, 128), jnp.float32)
        blk = blk.at[:, hd].set(a_s[hd, o * 128:(o + 1) * 128])
        blk = blk.at[:, H + hd].set(a_d[hd, o * 128:(o + 1) * 128])
        cols.append(blk)
    return jnp.stack(cols)


def _gat_layer(xin, src, dst, W, a_s, a_d, H, C, sc_a, sc_b):
    aall = _build_aall(a_s, a_d, C, H)
    hc, al = _mm_alpha(xin, W, aall, C)
    bdw = _bd(al)
    as_t = [al[:, h] for h in range(H)]
    ad_t = [al[:, H + h] for h in range(H)]
    bd_t = [bdw[:, H + h] for h in range(H)]
    outs = sc_a(src, dst, *as_t, *ad_t, *bd_t)
    t_list = list(outs[:H])
    s_p = outs[H]
    s0 = [s_p[0, h] for h in range(H)]
    s1 = [s_p[1, h] for h in range(H)]
    hcs = [hc[c] for c in range(C)]
    outp = sc_b(src, dst, *t_list, *s0, *s1, *hcs)
    return outp


def kernel(x, edge_index, W1, a_src1, a_dst1, b1, g1, be1, rm1, rv1,
           W2, a_src2, a_dst2, b2, g2, be2, rm2, rv2,
           W3, a_src3, a_dst3, b3):
    loop = jnp.arange(NN, dtype=jnp.int32)
    src = jnp.concatenate([edge_index[0].astype(jnp.int32), loop])
    dst = jnp.concatenate([edge_index[1].astype(jnp.int32), loop])
    npad_e = ETP - src.shape[0]
    src = jnp.concatenate([src, jnp.zeros((npad_e,), jnp.int32)])
    dst = jnp.concatenate([dst, jnp.full((npad_e,), DUMMY, jnp.int32)])
    src = src.reshape(NTILES, NBATCH, EB)
    dst = dst.reshape(NTILES, NBATCH, EB)
    xp = jnp.pad(x, ((0, NPAD - NN), (0, 0)))

    sc_a4 = _make_sc_a(4)
    sc_b4 = _make_sc_b(4, 16)
    sc_a1 = _make_sc_a(1)
    sc_b1_4 = _make_sc_b(1, 4)
    sc_b1_1 = _make_sc_b(1, 1)

    zero = jnp.zeros((128,), jnp.float32)
    one = jnp.ones((128,), jnp.float32)

    outp1 = _gat_layer(xp, src, dst, W1, a_src1, a_dst1, 4, 16, sc_a4, sc_b4)
    y1 = _combine(outp1, b1.reshape(16, 128), g1.reshape(16, 128),
                  be1.reshape(16, 128), rm1.reshape(16, 128),
                  rv1.reshape(16, 128), 16, "bn_elu")

    outp2 = _gat_layer(y1, src, dst, W2, a_src2, a_dst2, 1, 4, sc_a1, sc_b1_4)
    y2 = _combine(outp2, b2.reshape(4, 128), g2.reshape(4, 128),
                  be2.reshape(4, 128), rm2.reshape(4, 128),
                  rv2.reshape(4, 128), 4, "bn_elu")

    outp3 = _gat_layer(y2, src, dst, W3, a_src3, a_dst3, 1, 1, sc_a1, sc_b1_1)
    y3 = _combine(outp3, b3.reshape(1, 128), one.reshape(1, 128),
                  zero.reshape(1, 128), zero.reshape(1, 128),
                  one.reshape(1, 128), 1, "logsoftmax")
    return y3[:NN]


# speedup vs baseline: 6.2958x; 1.0985x over previous
import functools

import jax
import jax.numpy as jnp
from jax import lax
from jax.experimental import pallas as pl
from jax.experimental.pallas import tpu as pltpu
from jax.experimental.pallas import tpu_sc as plsc

NN = 10000
NPAD = 10240
EB = 128
NBATCH = 42
EPT = EB * NBATCH
NTILES = 32
ETP = EPT * NTILES
DUMMY = 10010
SLICE = NPAD // 16
EPS = 1e-5
NC = 2


def _mm_alpha_body(x_ref, w_ref, a_ref, h_ref, al_ref):
    h = jnp.dot(x_ref[...], w_ref[...], preferred_element_type=jnp.float32)
    h_ref[0] = h

    @pl.when(pl.program_id(1) == 0)
    def _():
        al_ref[...] = jnp.zeros_like(al_ref)

    al_ref[...] += jnp.dot(h, a_ref[0], preferred_element_type=jnp.float32)


def _mm_alpha(xp, W, aall, C):
    K = xp.shape[1]
    BM = 512
    return pl.pallas_call(
        _mm_alpha_body,
        grid=(NPAD // BM, C),
        in_specs=[
            pl.BlockSpec((BM, K), lambda m, c: (m, 0)),
            pl.BlockSpec((K, 128), lambda m, c: (0, c)),
            pl.BlockSpec((1, 128, 128), lambda m, c: (c, 0, 0)),
        ],
        out_specs=[
            pl.BlockSpec((1, BM, 128), lambda m, c: (c, m, 0)),
            pl.BlockSpec((BM, 128), lambda m, c: (m, 0)),
        ],
        out_shape=[
            jax.ShapeDtypeStruct((C, NPAD, 128), jnp.float32),
            jax.ShapeDtypeStruct((NPAD, 128), jnp.float32),
        ],
    )(xp, W, aall)


def _bd_body(a_ref, bd_ref):
    a = a_ref[...]
    ms = jnp.max(a)
    v = a + ms
    bd_ref[...] = jnp.maximum(v, 0.2 * v)


def _bd(al):
    return pl.pallas_call(
        _bd_body,
        out_shape=jax.ShapeDtypeStruct((NPAD, 128), jnp.float32),
    )(al)


def _combine_body(op_ref, b_ref, g_ref, be_ref, rm_ref, rv_ref, o_ref, *, act):
    o = op_ref[0, 0, 0] + op_ref[1, 0, 0]
    o = o + b_ref[0, 0]
    if act == "bn_elu":
        o = (o - rm_ref[0, 0]) / jnp.sqrt(rv_ref[0, 0] + EPS) * g_ref[0, 0] + be_ref[0, 0]
        o = jnp.where(o > 0, o, jnp.exp(o) - 1.0)
        rows = lax.broadcasted_iota(jnp.int32, o.shape, 0) + pl.program_id(1) * ASL
        o = jnp.where(rows < NN, o, 0.0)
    else:
        mx = jnp.max(o, axis=1, keepdims=True)
        s = o - mx
        o = s - jnp.log(jnp.sum(jnp.exp(s), axis=1, keepdims=True))
    o_ref[0] = o


def _combine(outp, bias, g, be, rm, rv, C, act):
    body = functools.partial(_combine_body, act=act)
    y = pl.pallas_call(
        body,
        grid=(C, 16),
        in_specs=[
            pl.BlockSpec((2, 1, 1, ASL, 128), lambda c, s: (0, c, s, 0, 0)),
            pl.BlockSpec((1, 1, 128), lambda c, s: (c, 0, 0)),
            pl.BlockSpec((1, 1, 128), lambda c, s: (c, 0, 0)),
            pl.BlockSpec((1, 1, 128), lambda c, s: (c, 0, 0)),
            pl.BlockSpec((1, 1, 128), lambda c, s: (c, 0, 0)),
            pl.BlockSpec((1, 1, 128), lambda c, s: (c, 0, 0)),
        ],
        out_specs=pl.BlockSpec((1, ASL, 128), lambda c, s: (s, 0, c)),
        out_shape=jax.ShapeDtypeStruct((16, ASL, C * 128), jnp.float32),
    )(outp, bias.reshape(C, 1, 128), g.reshape(C, 1, 128),
      be.reshape(C, 1, 128), rm.reshape(C, 1, 128), rv.reshape(C, 1, 128))
    return y.reshape(NACC, C * 128)


def _make_sc_a(H):
    mesh = plsc.VectorSubcoreMesh(core_axis_name="c", subcore_axis_name="s")
    out_type = [jax.ShapeDtypeStruct((NTILES, NBATCH, EB), jnp.float32) for _ in range(H)]
    out_type.append(jax.ShapeDtypeStruct((NC, H, NPAD), jnp.float32))
    scratch = [
        pltpu.VMEM((NBATCH, EB), jnp.int32),
        pltpu.VMEM((NBATCH, EB), jnp.int32),
        pltpu.VMEM((EB,), jnp.float32),
        pltpu.VMEM((EB,), jnp.float32),
        pltpu.VMEM((EB,), jnp.float32),
        pltpu.VMEM((EB,), jnp.float32),
        pltpu.VMEM((SLICE,), jnp.float32),
    ] + [pltpu.VMEM_SHARED((NPAD,), jnp.float32) for _ in range(H)]

    def body(*refs):
        src_h, dst_h = refs[0], refs[1]
        as_t = refs[2:2 + H]
        ad_t = refs[2 + H:2 + 2 * H]
        bd_t = refs[2 + 2 * H:2 + 3 * H]
        t_out = refs[2 + 3 * H:2 + 4 * H]
        s_out = refs[2 + 4 * H]
        src_v, dst_v, bs, bdv, bb, tb, zb = refs[3 + 4 * H:10 + 4 * H]
        s_sh = refs[10 + 4 * H:10 + 5 * H]
        core = lax.axis_index("c")
        sid = lax.axis_index("s")
        wid = sid * NC + core
        pltpu.sync_copy(src_h.at[wid], src_v)
        pltpu.sync_copy(dst_h.at[wid], dst_v)
        for j in range(SLICE // 16):
            zb[pl.ds(j * 16, 16)] = jnp.zeros((16,), jnp.float32)
        for h in range(H):
            pltpu.sync_copy(zb, s_sh[h].at[pl.ds(sid * SLICE, SLICE)])
        plsc.subcore_barrier()

        def bstep(b, carry):
            for h in range(H):
                pltpu.sync_copy(as_t[h].at[src_v.at[b]], bs)
                pltpu.sync_copy(ad_t[h].at[dst_v.at[b]], bdv)
                pltpu.sync_copy(bd_t[h].at[dst_v.at[b]], bb)
                for k in range(EB // 16):
                    sl = pl.ds(k * 16, 16)
                    e = bs[sl] + bdv[sl]
                    e = jnp.maximum(e, 0.2 * e)
                    tb[sl] = jnp.exp(e - bb[sl])
                pltpu.sync_copy(tb, t_out[h].at[wid, b])
                pltpu.sync_copy(tb, s_sh[h].at[dst_v.at[b]], add=True)
            return carry

        lax.fori_loop(0, NBATCH, bstep, 0)
        plsc.subcore_barrier()
        for h in range(H):
            pltpu.sync_copy(
                s_sh[h].at[pl.ds(sid * SLICE, SLICE)],
                s_out.at[core, h, pl.ds(sid * SLICE, SLICE)],
            )

    return pl.kernel(body, out_type=out_type, mesh=mesh, scratch_types=scratch)


NACC = 10016
ASL = NACC // 16


def _make_sc_b(H, C, c0, cn):
    mesh = plsc.VectorSubcoreMesh(core_axis_name="c", subcore_axis_name="s")
    out_type = jax.ShapeDtypeStruct((NC, cn, 16, ASL, 128), jnp.float32)
    scratch = [
        pltpu.VMEM((NBATCH, EB), jnp.int32),
        pltpu.VMEM((NBATCH, EB), jnp.int32),
        pltpu.VMEM((NBATCH * EB,), jnp.float32),
        pltpu.VMEM((EB, 128), jnp.float32),
        pltpu.VMEM((EB, 128), jnp.float32),
        pltpu.VMEM_SHARED((NACC, 128), jnp.float32),
        pltpu.SemaphoreType.DMA,
        pltpu.SemaphoreType.DMA,
        pltpu.SemaphoreType.DMA,
        pltpu.SemaphoreType.DMA,
    ]

    def body(*refs):
        src_h, dst_h = refs[0], refs[1]
        t_in = refs[2:2 + H]
        s0 = refs[2 + H:2 + 2 * H]
        s1 = refs[2 + 2 * H:2 + 3 * H]
        hcs = refs[2 + 3 * H:2 + 3 * H + cn]
        outp = refs[2 + 3 * H + cn]
        (src_v, dst_v, rv, gb0, gb1, acc,
         gsem0, gsem1, ssem0, ssem1) = refs[3 + 3 * H + cn:]
        gb = (gb0, gb1)
        gsem = (gsem0, gsem1)
        ssem = (ssem0, ssem1)
        core = lax.axis_index("c")
        sid = lax.axis_index("s")
        wid = sid * NC + core
        pltpu.sync_copy(src_h.at[wid], src_v)
        pltpu.sync_copy(dst_h.at[wid], dst_v)

        def zero_gbuf():
            def zstep(j, carry):
                for k in range(8):
                    gb0[j, pl.ds(k * 16, 16)] = jnp.zeros((16,), jnp.float32)
                return carry

            lax.fori_loop(0, EB, zstep, 0)

        for ci in range(cn):
            c = c0 + ci
            hd = c // (C // H)
            if ci == 0 or c % (C // H) == 0:
                def rstep(b, carry, hd=hd):
                    pltpu.sync_copy(t_in[hd].at[wid, b], gb1.at[0])
                    pltpu.sync_copy(s0[hd].at[dst_v.at[b]], gb1.at[1])
                    pltpu.sync_copy(s1[hd].at[dst_v.at[b]], gb1.at[2])
                    for k in range(EB // 16):
                        sl = pl.ds(k * 16, 16)
                        rv[pl.ds(b * EB + k * 16, 16)] = (
                            gb1[0, sl] / (gb1[1, sl] + gb1[2, sl] + 1e-16))
                    return carry

                lax.fori_loop(0, NBATCH, rstep, 0)
            zero_gbuf()
            for j in range(ASL // 128):
                pltpu.sync_copy(gb0, acc.at[pl.ds(sid * ASL + j * 128, 128)])
            if ASL % 128:
                pltpu.sync_copy(gb0.at[pl.ds(0, ASL % 128)],
                                acc.at[pl.ds(sid * ASL + (ASL // 128) * 128, ASL % 128)])
            plsc.subcore_barrier()

            def scale_buf(buf, b):
                def scale(g, c3):
                    rvec = rv[pl.ds(b * EB + g * 16, 16)]
                    for j in range(16):
                        s = rvec[j]
                        for k in range(8):
                            sl = pl.ds(k * 16, 16)
                            buf[g * 16 + j, sl] = buf[g * 16 + j, sl] * s
                    return c3

                lax.fori_loop(0, EB // 16, scale, 0)

            pltpu.async_copy(hcs[ci].at[src_v.at[0]], gb0, gsem0)

            def estep2(bb, carry, ci=ci):
                b0 = 2 * bb
                pltpu.make_async_copy(hcs[ci].at[src_v.at[b0]], gb0, gsem0).wait()
                pltpu.async_copy(hcs[ci].at[src_v.at[b0 + 1]], gb1, gsem1)
                scale_buf(gb0, b0)
                pltpu.sync_copy(gb0, acc.at[dst_v.at[b0]], add=True)
                pltpu.make_async_copy(hcs[ci].at[src_v.at[b0 + 1]], gb1, gsem1).wait()
                pltpu.async_copy(hcs[ci].at[src_v.at[b0 + 2]], gb0, gsem0)
                scale_buf(gb1, b0 + 1)
                pltpu.sync_copy(gb1, acc.at[dst_v.at[b0 + 1]], add=True)
                return carry

            lax.fori_loop(0, NBATCH // 2 - 1, estep2, 0)
            b0 = NBATCH - 2
            pltpu.make_async_copy(hcs[ci].at[src_v.at[b0]], gb0, gsem0).wait()
            pltpu.async_copy(hcs[ci].at[src_v.at[b0 + 1]], gb1, gsem1)
            scale_buf(gb0, b0)
            pltpu.sync_copy(gb0, acc.at[dst_v.at[b0]], add=True)
            pltpu.make_async_copy(hcs[ci].at[src_v.at[b0 + 1]], gb1, gsem1).wait()
            scale_buf(gb1, b0 + 1)
            pltpu.sync_copy(gb1, acc.at[dst_v.at[b0 + 1]], add=True)
            plsc.subcore_barrier()
            pltpu.sync_copy(
                acc.at[pl.ds(sid * ASL, ASL)],
                outp.at[core, ci, sid],
            )

    return pl.kernel(body, out_type=out_type, mesh=mesh, scratch_types=scratch)


def _build_aall(a_s, a_d, C, H):
    per = C // H
    cols = []
    for c in range(C):
        hd = c // per
        o = c % per
        blk = jnp.zeros((128, 128), jnp.float32)
        blk = blk.at[:, hd].set(a_s[hd, o * 128:(o + 1) * 128])
        blk = blk.at[:, H + hd].set(a_d[hd, o * 128:(o + 1) * 128])
        cols.append(blk)
    return jnp.stack(cols)


def _gat_layer(xin, src, dst, W, a_s, a_d, H, C, sc_a, sc_b_parts):
    aall = _build_aall(a_s, a_d, C, H)
    hc, al = _mm_alpha(xin, W, aall, C)
    bdw = _bd(al)
    as_t = [al[:, h] for h in range(H)]
    ad_t = [al[:, H + h] for h in range(H)]
    bd_t = [bdw[:, H + h] for h in range(H)]
    outs = sc_a(src, dst, *as_t, *ad_t, *bd_t)
    t_list = list(outs[:H])
    s_p = outs[H]
    s0 = [s_p[0, h] for h in range(H)]
    s1 = [s_p[1, h] for h in range(H)]
    parts = []
    for sc_b, c0, cn in sc_b_parts:
        hcs = [hc[c] for c in range(c0, c0 + cn)]
        parts.append(sc_b(src, dst, *t_list, *s0, *s1, *hcs))
    return jnp.concatenate(parts, axis=1) if len(parts) > 1 else parts[0]


def kernel(x, edge_index, W1, a_src1, a_dst1, b1, g1, be1, rm1, rv1,
           W2, a_src2, a_dst2, b2, g2, be2, rm2, rv2,
           W3, a_src3, a_dst3, b3):
    loop = jnp.arange(NN, dtype=jnp.int32)
    src = jnp.concatenate([edge_index[0].astype(jnp.int32), loop])
    dst = jnp.concatenate([edge_index[1].astype(jnp.int32), loop])
    npad_e = ETP - src.shape[0]
    src = jnp.concatenate([src, jnp.zeros((npad_e,), jnp.int32)])
    dst = jnp.concatenate([dst, jnp.full((npad_e,), DUMMY, jnp.int32)])
    src = src.reshape(NTILES, NBATCH, EB)
    dst = dst.reshape(NTILES, NBATCH, EB)
    xp = jnp.pad(x, ((0, NPAD - NN), (0, 0)))

    sc_a4 = _make_sc_a(4)
    sc_b4_parts = [(_make_sc_b(4, 16, c0, 4), c0, 4) for c0 in (0, 4, 8, 12)]
    sc_a1 = _make_sc_a(1)
    sc_b1_4 = [(_make_sc_b(1, 4, 0, 4), 0, 4)]
    sc_b1_1 = [(_make_sc_b(1, 1, 0, 1), 0, 1)]

    zero = jnp.zeros((128,), jnp.float32)
    one = jnp.ones((128,), jnp.float32)

    outp1 = _gat_layer(xp, src, dst, W1, a_src1, a_dst1, 4, 16, sc_a4, sc_b4_parts)
    y1 = _combine(outp1, b1.reshape(16, 128), g1.reshape(16, 128),
                  be1.reshape(16, 128), rm1.reshape(16, 128),
                  rv1.reshape(16, 128), 16, "bn_elu")

    y1 = jnp.pad(y1, ((0, NPAD - NACC), (0, 0)))
    outp2 = _gat_layer(y1, src, dst, W2, a_src2, a_dst2, 1, 4, sc_a1, sc_b1_4)
    y2 = _combine(outp2, b2.reshape(4, 128), g2.reshape(4, 128),
                  be2.reshape(4, 128), rm2.reshape(4, 128),
                  rv2.reshape(4, 128), 4, "bn_elu")

    y2 = jnp.pad(y2, ((0, NPAD - NACC), (0, 0)))
    outp3 = _gat_layer(y2, src, dst, W3, a_src3, a_dst3, 1, 1, sc_a1, sc_b1_1)
    y3 = _combine(outp3, b3.reshape(1, 128), one.reshape(1, 128),
                  zero.reshape(1, 128), zero.reshape(1, 128),
                  one.reshape(1, 128), 1, "logsoftmax")
    return y3[:NN]
